# baseline (speedup 1.0000x reference)
import functools

import jax
import jax.numpy as jnp
from jax import lax
from jax.experimental import pallas as pl
from jax.experimental.pallas import tpu as pltpu
from jax.experimental.pallas import tpu_sc as plsc

_DIM = 128
_NC = 2
_NS = 16
_NW = _NC * _NS
_CHUNK = 128
_TC_CH = 1024


def _sc_gather_fn(b):
    e_rows_w = 4 * b // _NW
    r_rows_w = 2 * b // _NW
    rows_w = e_rows_w + r_rows_w
    e_chunks = e_rows_w // _CHUNK
    r_chunks = r_rows_w // _CHUNK
    mesh = plsc.VectorSubcoreMesh(core_axis_name="c", subcore_axis_name="s")

    @functools.partial(
        pl.kernel,
        out_type=jax.ShapeDtypeStruct((_NW * rows_w, _DIM), jnp.float32),
        mesh=mesh,
        scratch_types=[
            pltpu.VMEM((rows_w,), jnp.int32),
            pltpu.VMEM((rows_w, _DIM), jnp.float32),
            pltpu.SemaphoreType.DMA,
            pltpu.SemaphoreType.DMA,
        ],
    )
    def gather(ent_hbm, rel_hbm, idx_hbm, out_hbm, idx_v, rows_v, gsem, osem):
        wid = lax.axis_index("s") * _NC + lax.axis_index("c")
        pltpu.sync_copy(idx_hbm.at[pl.ds(wid * rows_w, rows_w)], idx_v)
        gathers = []
        for j in range(e_chunks):
            gathers.append(pltpu.async_copy(
                ent_hbm.at[idx_v.at[pl.ds(j * _CHUNK, _CHUNK)]],
                rows_v.at[pl.ds(j * _CHUNK, _CHUNK)], gsem))
        for j in range(r_chunks):
            gathers.append(pltpu.async_copy(
                rel_hbm.at[idx_v.at[pl.ds((e_chunks + j) * _CHUNK, _CHUNK)]],
                rows_v.at[pl.ds((e_chunks + j) * _CHUNK, _CHUNK)], gsem))
        for g in gathers:
            g.wait()
        pltpu.async_copy(
            rows_v, out_hbm.at[pl.ds(wid * rows_w, rows_w)], osem).wait()

    return gather


def _unit(x):
    s = jnp.sum(x * x, axis=1)
    inv = lax.rsqrt(jnp.maximum(s, 1e-24))
    return x * inv[:, None]


def _tc_loss_fn(scale, ch):
    def _tc_loss(hp_ref, tp_ref, hn_ref, tn_ref, rp_ref, rn_ref, out_ref):
        i = pl.program_id(0)
        hp = hp_ref[...].reshape(ch, _DIM)
        tp = tp_ref[...].reshape(ch, _DIM)
        hn = hn_ref[...].reshape(ch, _DIM)
        tn = tn_ref[...].reshape(ch, _DIM)
        rp = rp_ref[...].reshape(ch, _DIM)
        rn = rn_ref[...].reshape(ch, _DIM)
        dp = _unit(hp) + _unit(rp) - _unit(tp)
        dn = _unit(hn) + _unit(rn) - _unit(tn)
        sp = jnp.maximum(jnp.sum(dp * dp, axis=1), 1e-30)
        sn = jnp.maximum(jnp.sum(dn * dn, axis=1), 1e-30)
        ep = sp * lax.rsqrt(sp)
        en = sn * lax.rsqrt(sn)
        part = jnp.sum(jnp.maximum(1.0 + ep - en, 0.0))

        @pl.when(i == 0)
        def _():
            out_ref[...] = jnp.zeros((1, 1), jnp.float32)

        out_ref[...] += part.reshape(1, 1)

        @pl.when(i == pl.num_programs(0) - 1)
        def _():
            out_ref[...] *= scale

    return _tc_loss


@jax.jit
def kernel(pos_triples, neg_triples, ent_emb, rel_emb):
    b = pos_triples.shape[0]
    e_rows_w = 4 * b // _NW
    r_rows_w = 2 * b // _NW
    rows_w = e_rows_w + r_rows_w
    n_slots = rows_w // _CHUNK

    idx_ent = jnp.concatenate([
        pos_triples[:, 0], pos_triples[:, 2],
        neg_triples[:, 0], neg_triples[:, 2],
    ])
    idx_rel = jnp.concatenate([
        pos_triples[:, 1], neg_triples[:, 1],
    ])
    idx_all = jnp.concatenate([
        idx_ent.reshape(_NW, e_rows_w),
        idx_rel.reshape(_NW, r_rows_w),
    ], axis=1).reshape(-1)

    rows = _sc_gather_fn(b)(ent_emb, rel_emb, idx_all)
    rows4 = rows.reshape(_NW, n_slots, _CHUNK, _DIM)

    ew = _TC_CH // e_rows_w
    rw = _TC_CH // r_rows_w
    nsteps = b // _TC_CH
    e_chunks = e_rows_w // _CHUNK
    r_chunks = r_rows_w // _CHUNK

    def espec(seg):
        return pl.BlockSpec(
            (ew, e_chunks, _CHUNK, _DIM),
            lambda i, seg=seg: (seg * nsteps + i, 0, 0, 0))

    def rspec(seg):
        return pl.BlockSpec(
            (rw, r_chunks, _CHUNK, _DIM),
            lambda i, seg=seg: (seg * nsteps + i, e_chunks // r_chunks,
                                0, 0))

    out = pl.pallas_call(
        _tc_loss_fn(1.0 / b, _TC_CH),
        grid=(nsteps,),
        in_specs=[espec(0), espec(1), espec(2), espec(3),
                  rspec(0), rspec(1)],
        out_specs=pl.BlockSpec((1, 1), lambda i: (0, 0)),
        out_shape=jax.ShapeDtypeStruct((1, 1), jnp.float32),
    )(rows4, rows4, rows4, rows4, rows4, rows4)
    return out[0, 0]


# speedup vs baseline: 1.0045x; 1.0045x over previous
import functools

import jax
import jax.numpy as jnp
from jax import lax
from jax.experimental import pallas as pl
from jax.experimental.pallas import tpu as pltpu
from jax.experimental.pallas import tpu_sc as plsc

_DIM = 128
_NC = 2
_NS = 16
_NW = _NC * _NS
_CHUNK = 128
_TC_CH = 2048


def _sc_gather_fn(b):
    e_rows_w = 4 * b // _NW
    r_rows_w = 2 * b // _NW
    rows_w = e_rows_w + r_rows_w
    e_chunks = e_rows_w // _CHUNK
    r_chunks = r_rows_w // _CHUNK
    mesh = plsc.VectorSubcoreMesh(core_axis_name="c", subcore_axis_name="s")

    @functools.partial(
        pl.kernel,
        out_type=jax.ShapeDtypeStruct((_NW * rows_w, _DIM), jnp.float32),
        mesh=mesh,
        scratch_types=[
            pltpu.VMEM((rows_w,), jnp.int32),
            pltpu.VMEM((rows_w, _DIM), jnp.float32),
            pltpu.SemaphoreType.DMA,
            pltpu.SemaphoreType.DMA,
        ],
    )
    def gather(ent_hbm, rel_hbm, idx_hbm, out_hbm, idx_v, rows_v, gsem, osem):
        wid = lax.axis_index("s") * _NC + lax.axis_index("c")
        pltpu.sync_copy(idx_hbm.at[pl.ds(wid * rows_w, rows_w)], idx_v)
        gathers = []
        for j in range(e_chunks):
            gathers.append(pltpu.async_copy(
                ent_hbm.at[idx_v.at[pl.ds(j * _CHUNK, _CHUNK)]],
                rows_v.at[pl.ds(j * _CHUNK, _CHUNK)], gsem))
        for j in range(r_chunks):
            gathers.append(pltpu.async_copy(
                rel_hbm.at[idx_v.at[pl.ds((e_chunks + j) * _CHUNK, _CHUNK)]],
                rows_v.at[pl.ds((e_chunks + j) * _CHUNK, _CHUNK)], gsem))
        for g in gathers:
            g.wait()
        pltpu.async_copy(
            rows_v, out_hbm.at[pl.ds(wid * rows_w, rows_w)], osem).wait()

    return gather


def _unit(x):
    s = jnp.sum(x * x, axis=1)
    inv = lax.rsqrt(jnp.maximum(s, 1e-24))
    return x * inv[:, None]


def _tc_loss_fn(scale, ch):
    def _tc_loss(hp_ref, tp_ref, hn_ref, tn_ref, rp_ref, rn_ref, out_ref):
        i = pl.program_id(0)
        hp = hp_ref[...].reshape(ch, _DIM)
        tp = tp_ref[...].reshape(ch, _DIM)
        hn = hn_ref[...].reshape(ch, _DIM)
        tn = tn_ref[...].reshape(ch, _DIM)
        rp = rp_ref[...].reshape(ch, _DIM)
        rn = rn_ref[...].reshape(ch, _DIM)
        dp = _unit(hp) + _unit(rp) - _unit(tp)
        dn = _unit(hn) + _unit(rn) - _unit(tn)
        sp = jnp.maximum(jnp.sum(dp * dp, axis=1), 1e-30)
        sn = jnp.maximum(jnp.sum(dn * dn, axis=1), 1e-30)
        ep = sp * lax.rsqrt(sp)
        en = sn * lax.rsqrt(sn)
        part = jnp.sum(jnp.maximum(1.0 + ep - en, 0.0))

        @pl.when(i == 0)
        def _():
            out_ref[...] = jnp.zeros((1, 1), jnp.float32)

        out_ref[...] += part.reshape(1, 1)

        @pl.when(i == pl.num_programs(0) - 1)
        def _():
            out_ref[...] *= scale

    return _tc_loss


@jax.jit
def kernel(pos_triples, neg_triples, ent_emb, rel_emb):
    b = pos_triples.shape[0]
    e_rows_w = 4 * b // _NW
    r_rows_w = 2 * b // _NW
    rows_w = e_rows_w + r_rows_w
    n_slots = rows_w // _CHUNK

    idx_ent = jnp.concatenate([
        pos_triples[:, 0], pos_triples[:, 2],
        neg_triples[:, 0], neg_triples[:, 2],
    ])
    idx_rel = jnp.concatenate([
        pos_triples[:, 1], neg_triples[:, 1],
    ])
    idx_all = jnp.concatenate([
        idx_ent.reshape(_NW, e_rows_w),
        idx_rel.reshape(_NW, r_rows_w),
    ], axis=1).reshape(-1)

    rows = _sc_gather_fn(b)(ent_emb, rel_emb, idx_all)
    rows4 = rows.reshape(_NW, n_slots, _CHUNK, _DIM)

    ew = _TC_CH // e_rows_w
    rw = _TC_CH // r_rows_w
    nsteps = b // _TC_CH
    e_chunks = e_rows_w // _CHUNK
    r_chunks = r_rows_w // _CHUNK

    def espec(seg):
        return pl.BlockSpec(
            (ew, e_chunks, _CHUNK, _DIM),
            lambda i, seg=seg: (seg * nsteps + i, 0, 0, 0))

    def rspec(seg):
        return pl.BlockSpec(
            (rw, r_chunks, _CHUNK, _DIM),
            lambda i, seg=seg: (seg * nsteps + i, e_chunks // r_chunks,
                                0, 0))

    out = pl.pallas_call(
        _tc_loss_fn(1.0 / b, _TC_CH),
        grid=(nsteps,),
        in_specs=[espec(0), espec(1), espec(2), espec(3),
                  rspec(0), rspec(1)],
        out_specs=pl.BlockSpec((1, 1), lambda i: (0, 0)),
        out_shape=jax.ShapeDtypeStruct((1, 1), jnp.float32),
    )(rows4, rows4, rows4, rows4, rows4, rows4)
    return out[0, 0]
